# baseline (speedup 1.0000x reference)
import math

import jax
import jax.numpy as jnp
import numpy as np
from jax import lax
from jax.experimental import pallas as pl
from jax.experimental.pallas import tpu as pltpu
from jax.experimental.pallas import tpu_sc as plsc

_D_MODEL = 128
_SEQ = 200
_BATCH = 1024
_SCALE = math.sqrt(float(_D_MODEL))

_NUM_CORES = 2
_NUM_SUBCORES = 16
_NW = _NUM_CORES * _NUM_SUBCORES
_VREGS_PER_ROW = _D_MODEL // 16

_TCHUNKS = 8
_BCHUNKS = _NW // _TCHUNKS
_TPW = _SEQ // _TCHUNKS
_BPW = _BATCH // _BCHUNKS
_NBANK = 3
_NGROUPS = _TPW // _NBANK


def _make_pe():
    pe = np.zeros((_SEQ, _D_MODEL), dtype=np.float32)
    position = np.arange(0, _SEQ, dtype=np.float32)[:, None]
    div_term = np.exp(
        np.arange(0, _D_MODEL, 2, dtype=np.float32)
        * -(math.log(10000.0) / _D_MODEL)
    )
    pe[:, 0::2] = np.sin(position * div_term)
    pe[:, 1::2] = np.cos(position * div_term)
    return pe


_PE = _make_pe()


def _body(lut_hbm, idxt_hbm, pe_hbm, out_hbm, *scr):
    idxb = scr[0:_NBANK]
    rows = scr[_NBANK:2 * _NBANK]
    pe_v = scr[2 * _NBANK]
    isem = scr[2 * _NBANK + 1:2 * _NBANK + 1 + _NBANK]
    gsem = scr[2 * _NBANK + 1 + _NBANK:2 * _NBANK + 1 + 2 * _NBANK]
    ssem = scr[2 * _NBANK + 1 + 2 * _NBANK:2 * _NBANK + 1 + 3 * _NBANK]
    wid = lax.axis_index("s") * _NUM_CORES + lax.axis_index("c")
    t0 = (wid % _TCHUNKS) * _TPW
    b0 = pl.multiple_of((wid // _TCHUNKS) * _BPW, _BPW)
    pltpu.sync_copy(
        pe_hbm.at[pl.ds(pl.multiple_of(t0 * _D_MODEL, 8), _TPW * _D_MODEL)],
        pe_v)

    def fire_idx(ct, p):
        pltpu.async_copy(
            idxt_hbm.at[
                pl.ds(pl.multiple_of((t0 + ct) * _BATCH + b0, 8), _BPW)],
            idxb[p], isem[p])

    def wait_idx(p):
        pltpu.make_async_copy(
            idxt_hbm.at[pl.ds(0, _BPW)], idxb[p], isem[p]).wait()

    def fire_gather(p):
        pltpu.async_copy(lut_hbm.at[idxb[p]], rows[p], gsem[p])

    def wait_gather(p):
        pltpu.make_async_copy(lut_hbm.at[idxb[p]], rows[p], gsem[p]).wait()

    def fire_store(ct, p):
        pltpu.async_copy(
            rows[p],
            out_hbm.at[
                pl.ds(b0, _BPW),
                pl.ds(pl.multiple_of((t0 + ct) * _D_MODEL, _D_MODEL),
                      _D_MODEL)],
            ssem[p])

    def wait_store(p):
        pltpu.make_async_copy(
            rows[p],
            out_hbm.at[pl.ds(0, _BPW), pl.ds(0, _D_MODEL)], ssem[p]).wait()

    def compute(ct, p):
        rb = rows[p]
        pes = [pe_v[pl.ds(ct * _D_MODEL + jj * 16, 16)]
               for jj in range(_VREGS_PER_ROW)]

        @plsc.parallel_loop(0, _BPW, unroll=4)
        def _row_loop(r):
            for jj in range(_VREGS_PER_ROW):
                sl = pl.ds(jj * 16, 16)
                rb[r, sl] = rb[r, sl] * _SCALE + pes[jj]

    fire_idx(0, 0)
    fire_idx(1, 1)
    fire_idx(2, 2)
    wait_idx(0)
    fire_gather(0)
    wait_idx(1)
    fire_gather(1)

    @pl.loop(0, _NGROUPS)
    def _group(g):
        for b in range(_NBANK):
            q = 3 * g + b
            wait_gather(b)
            compute(q, b)
            fire_store(q, b)
            if b == 0:
                fire_idx(q + 3, b)
            else:
                @pl.when(g < _NGROUPS - 1)
                def _():
                    fire_idx(q + 3, b)
            c = (b + 2) % _NBANK
            if b < 2:
                if b == 0:
                    @pl.when(g >= 1)
                    def _():
                        wait_store(c)
                else:
                    wait_store(c)
                wait_idx(c)
                fire_gather(c)
            else:
                wait_store(c)

                @pl.when(g < _NGROUPS - 1)
                def _():
                    wait_idx(c)
                    fire_gather(c)

    wait_gather(0)
    compute(_TPW - 1, 0)
    fire_store(_TPW - 1, 0)
    wait_store(2)
    wait_store(0)


@jax.jit
def _run(lut, idxt, pe):
    kern = pl.kernel(
        _body,
        out_type=jax.ShapeDtypeStruct((_BATCH, _SEQ * _D_MODEL), jnp.float32),
        mesh=plsc.VectorSubcoreMesh(
            core_axis_name="c", subcore_axis_name="s",
            num_cores=_NUM_CORES, num_subcores=_NUM_SUBCORES,
        ),
        scratch_types=(
            [pltpu.VMEM((_BPW,), jnp.int32)] * _NBANK
            + [pltpu.VMEM((_BPW, _D_MODEL), jnp.float32)] * _NBANK
            + [pltpu.VMEM((_TPW * _D_MODEL,), jnp.float32)]
            + [pltpu.SemaphoreType.DMA] * (3 * _NBANK)
        ),
    )
    return kern(lut, idxt, pe)


def kernel(x, lut):
    idxt = x.astype(jnp.int32).T.reshape(-1)
    pe = jnp.asarray(_PE.reshape(-1))
    return _run(lut, idxt, pe).reshape(_BATCH, _SEQ, _D_MODEL)


# speedup vs baseline: 1.6420x; 1.6420x over previous
import math

import jax
import jax.numpy as jnp
import numpy as np
from jax import lax
from jax.experimental import pallas as pl
from jax.experimental.pallas import tpu as pltpu
from jax.experimental.pallas import tpu_sc as plsc

_D_MODEL = 128
_SEQ = 200
_BATCH = 1024
_SCALE = math.sqrt(float(_D_MODEL))

_NUM_CORES = 2
_NUM_SUBCORES = 16
_NW = _NUM_CORES * _NUM_SUBCORES
_SEQS_PER_W = _BATCH // _NW
_VREGS_PER_ROW = _D_MODEL // 16

_QUAD = 4
_NCHUNK = 5
_CHUNK = _SEQ // _NCHUNK
_ROWS_PER_SLOT = _QUAD * _CHUNK
_NBANK = 5
_KGROUPS = _SEQS_PER_W // _QUAD


def _make_pe():
    pe = np.zeros((_SEQ, _D_MODEL), dtype=np.float32)
    position = np.arange(0, _SEQ, dtype=np.float32)[:, None]
    div_term = np.exp(
        np.arange(0, _D_MODEL, 2, dtype=np.float32)
        * -(math.log(10000.0) / _D_MODEL)
    )
    pe[:, 0::2] = np.sin(position * div_term)
    pe[:, 1::2] = np.cos(position * div_term)
    return pe


_PE = _make_pe()


def _body(lut_hbm, idx_hbm, pe_hbm, out_hbm, *scr):
    idxb = scr[0:_NBANK]
    rows = scr[_NBANK:2 * _NBANK]
    pe_v = scr[2 * _NBANK]
    base_i = 2 * _NBANK + 1
    isem = scr[base_i:base_i + _NBANK]
    gsem = scr[base_i + _NBANK:base_i + 2 * _NBANK]
    ssem = scr[base_i + 2 * _NBANK:base_i + 3 * _NBANK]
    wid = lax.axis_index("s") * _NUM_CORES + lax.axis_index("c")
    wbase = wid * _SEQS_PER_W
    pltpu.sync_copy(pe_hbm, pe_v)

    def chunk_base(k, i, j):
        return (wbase + _QUAD * k + i) * _SEQ + _CHUNK * j

    def fire_idxs(k, j, a):
        for i in range(_QUAD):
            pltpu.async_copy(
                idx_hbm.at[pl.ds(chunk_base(k, i, j), _CHUNK)],
                idxb[a].at[pl.ds(_CHUNK * i, _CHUNK)], isem[a])

    def wait_idxs(a):
        for i in range(_QUAD):
            pltpu.make_async_copy(
                idx_hbm.at[pl.ds(0, _CHUNK)],
                idxb[a].at[pl.ds(_CHUNK * i, _CHUNK)], isem[a]).wait()

    def fire_gather(a):
        pltpu.async_copy(lut_hbm.at[idxb[a]], rows[a], gsem[a])

    def wait_gather(a):
        pltpu.make_async_copy(lut_hbm.at[idxb[a]], rows[a], gsem[a]).wait()

    def fire_stores(k, j, a):
        for i in range(_QUAD):
            pltpu.async_copy(
                rows[a].at[pl.ds(_CHUNK * i, _CHUNK)],
                out_hbm.at[pl.ds(chunk_base(k, i, j), _CHUNK)], ssem[a])

    def wait_stores(a):
        for i in range(_QUAD):
            pltpu.make_async_copy(
                rows[a].at[pl.ds(_CHUNK * i, _CHUNK)],
                out_hbm.at[pl.ds(0, _CHUNK)], ssem[a]).wait()

    def compute_quad(a, j):
        rb = rows[a]

        @plsc.parallel_loop(0, _CHUNK, unroll=2)
        def _row_loop(r):
            for jj in range(_VREGS_PER_ROW):
                sl = pl.ds(jj * 16, 16)
                pe_reg = pe_v[_CHUNK * j + r, sl]
                for i in range(_QUAD):
                    rb[_CHUNK * i + r, sl] = (
                        rb[_CHUNK * i + r, sl] * _SCALE + pe_reg)

    fire_idxs(0, 0, 0)
    fire_idxs(0, 1, 1)
    fire_idxs(0, 2, 2)
    wait_idxs(0)
    fire_gather(0)
    wait_idxs(1)
    fire_gather(1)

    @pl.loop(0, _KGROUPS)
    def _group(k):
        for j in range(_NCHUNK):
            a = j
            wait_gather(a)
            compute_quad(a, j)
            fire_stores(k, j, a)
            i3 = (j + 3) % _NCHUNK
            if j <= 1:
                fire_idxs(k, j + 3, i3)
            else:
                @pl.when(k < _KGROUPS - 1)
                def _():
                    fire_idxs(k + 1, (j + 3) % _NCHUNK, i3)
            g2 = (j + 2) % _NCHUNK
            if j <= 2:
                @pl.when(k >= 1)
                def _():
                    wait_stores(g2)
                wait_idxs(g2)
                fire_gather(g2)
            else:
                wait_stores(g2)

                @pl.when(k < _KGROUPS - 1)
                def _():
                    wait_idxs(g2)
                    fire_gather(g2)

    wait_stores(2)
    wait_stores(3)
    wait_stores(4)


@jax.jit
def _run(lut, idx, pe):
    kern = pl.kernel(
        _body,
        out_type=jax.ShapeDtypeStruct((_BATCH * _SEQ, _D_MODEL), jnp.float32),
        mesh=plsc.VectorSubcoreMesh(
            core_axis_name="c", subcore_axis_name="s",
            num_cores=_NUM_CORES, num_subcores=_NUM_SUBCORES,
        ),
        scratch_types=(
            [pltpu.VMEM((_ROWS_PER_SLOT,), jnp.int32)] * _NBANK
            + [pltpu.VMEM((_ROWS_PER_SLOT, _D_MODEL), jnp.float32)] * _NBANK
            + [pltpu.VMEM((_SEQ, _D_MODEL), jnp.float32)]
            + [pltpu.SemaphoreType.DMA] * (3 * _NBANK)
        ),
    )
    return kern(lut, idx, pe)


def kernel(x, lut):
    idx = x.reshape(-1).astype(jnp.int32)
    pe = jnp.asarray(_PE)
    return _run(lut, idx, pe).reshape(_BATCH, _SEQ, _D_MODEL)
